# baseline (speedup 1.0000x reference)
import jax
import jax.numpy as jnp
from jax.experimental import pallas as pl
from jax.experimental.pallas import tpu as pltpu

NUM_EXPERTS = 16
HIDDEN = 64
FLAT = NUM_EXPERTS * HIDDEN
GPAD = 128


def _moe_body(x_ref, ew1_ref, gw1_ref, gw2_ref, bcat_ref, gb2_ref,
              ew2_ref, eb2_ref, out_ref, w1s_ref):
    b = pl.program_id(0)
    f = x_ref.shape[1]

    @pl.when(b == 0)
    def _prep():
        w1s_ref[:FLAT, :] = ew1_ref[...].astype(jnp.bfloat16)
        w1s_ref[FLAT:FLAT + HIDDEN, :] = gw1_ref[...].astype(jnp.bfloat16)
        w1s_ref[FLAT + HIDDEN:, :] = jnp.zeros(
            (GPAD - HIDDEN, w1s_ref.shape[1]), jnp.bfloat16)

    xb16 = x_ref[0].astype(jnp.bfloat16)

    ha = jnp.maximum(
        jax.lax.dot_general(xb16, w1s_ref[...], (((1,), (1,)), ((), ())),
                            preferred_element_type=jnp.float32)
        + bcat_ref[...], 0.0)
    h = ha[:, :FLAT]
    g1 = ha[:, FLAT:]

    logits = (jax.lax.dot_general(g1, gw2_ref[...], (((1,), (1,)), ((), ())),
                                  preferred_element_type=jnp.float32)
              + gb2_ref[...])
    m = jnp.max(logits, axis=1, keepdims=True)
    el = jnp.exp(logits - m)
    z = jnp.sum(el, axis=1, keepdims=True)

    m1 = jnp.max(el, axis=1, keepdims=True)
    el2 = jnp.where(el == m1, -1.0, el)
    m2 = jnp.max(el2, axis=1, keepdims=True)
    w = jnp.where(el >= m2, el, 0.0) / z

    c = jax.lax.dot_general(w, h, (((0,), (0,)), ((), ())),
                            preferred_element_type=jnp.float32)
    eidx = jax.lax.broadcasted_iota(jnp.int32, (NUM_EXPERTS, FLAT), 0)
    jidx = jax.lax.broadcasted_iota(jnp.int32, (NUM_EXPERTS, FLAT), 1)
    s = jnp.sum(jnp.where(jidx // HIDDEN == eidx, c, 0.0),
                axis=0, keepdims=True)
    wsum = jnp.sum(w, axis=0, keepdims=True)

    acc = jax.lax.dot_general(wsum, eb2_ref[...], (((1,), (0,)), ((), ())),
                              preferred_element_type=jnp.float32)
    for e in range(NUM_EXPERTS):
        acc = acc + jax.lax.dot_general(
            s[:, e * HIDDEN:(e + 1) * HIDDEN], ew2_ref[e],
            (((1,), (1,)), ((), ())),
            preferred_element_type=jnp.float32)
    out_ref[...] = (acc * (1.0 / f))[None]


def kernel(x, gw1, gb1, gw2, gb2, ew1, eb1, ew2, eb2):
    B, F, D = x.shape
    E, H, _ = ew1.shape
    O = ew2.shape[1]

    ew1r = ew1.reshape(E * H, D)
    gw2p = jnp.concatenate(
        [gw2, jnp.zeros((E, GPAD - H), gw2.dtype)], axis=1)
    bcat = jnp.concatenate(
        [eb1.reshape(1, E * H), gb1.reshape(1, H),
         jnp.zeros((1, GPAD - H), jnp.float32)], axis=1)
    gb2r = gb2.reshape(1, E)

    full = lambda *shape: pl.BlockSpec(shape, lambda b: (0,) * len(shape))
    out = pl.pallas_call(
        _moe_body,
        grid=(B,),
        in_specs=[
            pl.BlockSpec((1, F, D), lambda b: (b, 0, 0)),
            full(E * H, D), full(H, D), full(E, GPAD),
            full(1, FLAT + GPAD), full(1, E),
            full(E, O, H), full(E, O),
        ],
        out_specs=pl.BlockSpec((1, 1, O), lambda b: (b, 0, 0)),
        out_shape=jax.ShapeDtypeStruct((B, 1, O), x.dtype),
        scratch_shapes=[pltpu.VMEM((FLAT + GPAD, D), jnp.bfloat16)],
    )(x, ew1r, gw1, gw2p, bcat, gb2r, ew2, eb2)
    return out.reshape(B, 1, 1, O)
